# Initial kernel scaffold; baseline (speedup 1.0000x reference)
import jax
import jax.numpy as jnp
from jax.experimental import pallas as pl


def kernel(hidden_states, gate_w, expert_w):
    raise NotImplementedError("write your pallas kernel here")


# speedup vs baseline: 1.2501x; 1.2501x over previous
import jax
import jax.numpy as jnp
from jax import lax
from jax.experimental import pallas as pl
from jax.experimental.pallas import tpu as pltpu

_E = 8
_K = 2
_D = 1024


def _dense_moe_body(x_ref, w_ref, wc_ref, o_ref):
    e = pl.program_id(0)
    part = lax.dot_general(
        x_ref[...], w_ref[...],
        (((1,), (1,)), ((), ())),
        preferred_element_type=jnp.float32,
    )
    val = part * wc_ref[...]

    @pl.when(e == 0)
    def _init():
        o_ref[...] = val

    @pl.when(e > 0)
    def _acc():
        o_ref[...] += val


def kernel(hidden_states, gate_w, expert_w):
    B, S, D = hidden_states.shape
    T = B * S
    hs = hidden_states.reshape(-1, D)

    router_logits = hs @ gate_w.T
    probs = jax.nn.softmax(router_logits.astype(jnp.float32), axis=1)
    rw, sel = lax.top_k(probs, _K)
    rw = rw / jnp.sum(rw, axis=-1, keepdims=True)

    wmat = jnp.sum(
        rw[:, :, None] * (sel[:, :, None] == jnp.arange(_E)[None, None, :]),
        axis=1,
    ).astype(jnp.float32)

    x_bf = hs.astype(jnp.bfloat16)
    w_bf = expert_w.astype(jnp.bfloat16)
    wcol = wmat.T.reshape(_E, T, 1)

    out = pl.pallas_call(
        _dense_moe_body,
        grid=(_E,),
        in_specs=[
            pl.BlockSpec((T, D), lambda e: (0, 0)),
            pl.BlockSpec((None, D, D), lambda e: (e, 0, 0)),
            pl.BlockSpec((None, T, 1), lambda e: (e, 0, 0)),
        ],
        out_specs=pl.BlockSpec((T, D), lambda e: (0, 0)),
        out_shape=jax.ShapeDtypeStruct((T, D), jnp.float32),
    )(x_bf, w_bf, wcol)

    return out.reshape(B, S, D), router_logits
